# baseline (speedup 1.0000x reference)
import functools

import jax
import jax.numpy as jnp
from jax import lax
from jax.experimental import pallas as pl
from jax.experimental.pallas import tpu as pltpu
from jax.experimental.pallas import tpu_sc as plsc

D = 64
B = 4096
HIST = 50
NC, NS, L = 2, 16, 16
NW = NC * NS
BPW = B // NW
HPAD = 56
GROUP = 2 * HPAD
GROUPS = BPW // 2
NBUF = 8

_mesh = plsc.VectorSubcoreMesh(core_axis_name="c", subcore_axis_name="s")

ITEMS = 100001
CHUNK = 128
NFULL = ITEMS // CHUNK
TAIL = ITEMS - NFULL * CHUNK


@functools.partial(
    pl.kernel,
    mesh=_mesh,
    out_type=jax.ShapeDtypeStruct((ITEMS * D,), jnp.float32),
    scratch_types=(
        [pltpu.VMEM((D, CHUNK), jnp.float32) for _ in range(2)]
        + [pltpu.VMEM((CHUNK * D,), jnp.float32) for _ in range(2)]
        + [pltpu.VMEM((D, TAIL), jnp.float32),
           pltpu.VMEM((TAIL * D,), jnp.float32)]
        + [pltpu.SemaphoreType.DMA for _ in range(5)]
    ),
    compiler_params=pltpu.CompilerParams(use_tc_tiling_on_sc=True,
                                         needs_layout_passes=False),
)
def _relayout_sc(et_hbm, out_hbm, tin0, tin1, tout0, tout1, tin_p, tout_p,
                 sin0, sin1, sout0, sout1, sin_p):
    tins, touts = (tin0, tin1), (tout0, tout1)
    sins, souts = (sin0, sin1), (sout0, sout1)
    wid = lax.axis_index("s") * NC + lax.axis_index("c")
    lane = lax.iota(jnp.int32, L)

    def _in_copy(c, p):
        return pltpu.make_async_copy(
            et_hbm.at[:, pl.ds(c * CHUNK, CHUNK)], tins[p], sins[p])

    def _out_copy(c, p):
        return pltpu.make_async_copy(
            touts[p], out_hbm.at[pl.ds(c * (CHUNK * D), CHUNK * D)], souts[p])

    def _transpose(tin, tout, ncols):
        def body(il, carry):
            cols = jnp.full((L,), il, jnp.int32)
            for k in range(D // L):
                v = plsc.load_gather(tin, [lane + (k * L), cols])
                tout[pl.ds(il * D + k * L, L)] = v
            return carry
        lax.fori_loop(0, ncols, body, 0)

    for p in range(2):
        c0 = p * NW + wid

        @pl.when(c0 < NFULL)
        def _():
            _in_copy(c0, p).start()

    @pl.when(wid == NW - 1)
    def _():
        pltpu.make_async_copy(
            et_hbm.at[:, pl.ds(NFULL * CHUNK, TAIL)], tin_p, sin_p).start()

    def outer(jj, carry):
        for p in range(2):
            j = jj * 2 + p
            c = j * NW + wid
            valid = c < NFULL

            @pl.when(valid)
            def _():
                _in_copy(c, p).wait()

            @pl.when(valid & (j >= 2))
            def _():
                _out_copy(c - 2 * NW, p).wait()

            @pl.when(valid)
            def _():
                _transpose(tins[p], touts[p], CHUNK)
                _out_copy(c, p).start()

            @pl.when(c + 2 * NW < NFULL)
            def _():
                _in_copy(c + 2 * NW, p).start()
        return carry

    lax.fori_loop(0, (NFULL // NW) // 2 + 1, outer, 0)

    nfw = (NFULL - 1 - wid) // NW + 1
    for p in range(2):
        jp = nfw - 1 - ((nfw - 1 - p) % 2)
        _out_copy(jp * NW + wid, p).wait()

    @pl.when(wid == NW - 1)
    def _():
        pltpu.make_async_copy(
            et_hbm.at[:, pl.ds(NFULL * CHUNK, TAIL)], tin_p, sin_p).wait()
        _transpose(tin_p, tout_p, TAIL)
        pltpu.sync_copy(tout_p,
                        out_hbm.at[pl.ds(NFULL * (CHUNK * D), TAIL * D)])


@functools.partial(
    pl.kernel,
    mesh=_mesh,
    out_type=jax.ShapeDtypeStruct((B,), jnp.float32),
    scratch_types=(
        [
            pltpu.VMEM((HPAD, BPW), jnp.int32),
            pltpu.VMEM((GROUPS, GROUP), jnp.int32),
            pltpu.VMEM((BPW,), jnp.int32),
            pltpu.VMEM((BPW, D), jnp.float32),
            pltpu.VMEM((BPW,), jnp.float32),
        ]
        + [pltpu.VMEM((GROUP, D), jnp.float32) for _ in range(NBUF)]
        + [pltpu.SemaphoreType.DMA for _ in range(NBUF + 1)]
    ),
    compiler_params=pltpu.CompilerParams(use_tc_tiling_on_sc=False,
                                         needs_layout_passes=False),
)
def _bprmf_sc(seq_hbm, tgt_hbm, table_hbm, out_hbm,
              st, idx_v, tgt_idx, tgt_rows, out_buf, *rest):
    bufs = rest[:NBUF]
    sems = rest[NBUF:2 * NBUF]
    tsem = rest[2 * NBUF]

    wid = lax.axis_index("s") * NC + lax.axis_index("c")
    base = wid * BPW

    lane = lax.iota(jnp.int32, L)
    izero = jnp.zeros((L,), jnp.int32)

    pltpu.sync_copy(seq_hbm.at[:, pl.ds(base, BPW)], st.at[pl.ds(0, HIST)])
    pltpu.sync_copy(tgt_hbm.at[wid], tgt_idx)

    pltpu.async_copy(table_hbm.at[tgt_idx], tgt_rows, tsem)

    for r in range(HIST, HPAD):
        for c in range(BPW // L):
            st[r, pl.ds(c * L, L)] = izero

    def _transpose_group(gg):
        for r in range(2):
            cols = jnp.full((L,), 2 * gg + r, jnp.int32)
            for off in (0, L, 2 * L, HPAD - L):
                v = plsc.load_gather(st, [lane + off, cols])
                idx_v[gg, pl.ds(r * HPAD + off, L)] = v

    for b_ in range(NBUF):
        _transpose_group(b_)
    for b_ in range(NBUF):
        pltpu.async_copy(table_hbm.at[idx_v.at[b_]], bufs[b_], sems[b_])

    def _tr_body(gg, carry):
        _transpose_group(gg)
        return carry
    lax.fori_loop(NBUF, GROUPS, _tr_body, 0, unroll=4)

    pltpu.make_async_copy(table_hbm.at[tgt_idx], tgt_rows, tsem).wait()

    zero = jnp.zeros((L,), jnp.float32)
    one = jnp.ones((L,), jnp.float32)

    def _allreduce_sum(v):
        for k in (8, 4, 2, 1):
            v = v + v.at[lane ^ k].get(mode="promise_in_bounds")
        return v

    def _process(gg, buf, res, pos0):
        for r in range(2):
            rowbase = r * HPAD
            pos = pos0 + r

            def jbody(j, accs):
                a0, a1, a2, a3 = accs
                row = rowbase + j
                a0 = a0 + buf[row, pl.ds(0, L)]
                a1 = a1 + buf[row, pl.ds(L, L)]
                a2 = a2 + buf[row, pl.ds(2 * L, L)]
                a3 = a3 + buf[row, pl.ds(3 * L, L)]
                return (a0, a1, a2, a3)

            a0, a1, a2, a3 = lax.fori_loop(0, HIST, jbody,
                                           (zero, zero, zero, zero))

            rr = 2 * gg + r
            t0 = tgt_rows[rr, pl.ds(0, L)]
            t1 = tgt_rows[rr, pl.ds(L, L)]
            t2 = tgt_rows[rr, pl.ds(2 * L, L)]
            t3 = tgt_rows[rr, pl.ds(3 * L, L)]
            dotv = a0 * t0 + a1 * t1 + a2 * t2 + a3 * t3

            s0 = idx_v[gg, pl.ds(rowbase, L)]
            s1 = idx_v[gg, pl.ds(rowbase + L, L)]
            s2 = idx_v[gg, pl.ds(rowbase + 2 * L, L)]
            s3 = idx_v[gg, pl.ds(rowbase + HPAD - L, L)]
            w = (jnp.where(s0 != 0, one, zero)
                 + jnp.where(s1 != 0, one, zero)
                 + jnp.where(s2 != 0, one, zero)
                 + jnp.where((lane >= L // 2) & (s3 != 0), one, zero))
            pred_v = _allreduce_sum(dotv) / _allreduce_sum(w)
            res = jnp.where(lane == pos, pred_v, res)
        return res

    def outer(i, carry):
        res = zero
        for b_ in range(NBUF):
            gg = i * NBUF + b_
            pltpu.make_async_copy(table_hbm.at[idx_v.at[b_]],
                                  bufs[b_], sems[b_]).wait()
            res = _process(gg, bufs[b_], res, 2 * b_)

            @pl.when(gg + NBUF < GROUPS)
            def _():
                pltpu.async_copy(table_hbm.at[idx_v.at[gg + NBUF]],
                                 bufs[b_], sems[b_])
        out_buf[pl.ds(i * L, L)] = res
        return carry

    lax.fori_loop(0, GROUPS // NBUF, outer, 0)

    pltpu.sync_copy(out_buf, out_hbm.at[pl.ds(base, BPW)])


def kernel(seq, target, embed_weight):
    seq_t = jnp.swapaxes(seq.astype(jnp.int32), 0, 1)
    tgt_w = target.astype(jnp.int32).reshape(NW, BPW)
    return _bprmf_sc(seq_t, tgt_w, embed_weight)


# speedup vs baseline: 5.7265x; 5.7265x over previous
import functools

import jax
import jax.numpy as jnp
from jax import lax
from jax.experimental import pallas as pl
from jax.experimental.pallas import tpu as pltpu
from jax.experimental.pallas import tpu_sc as plsc

D = 64
B = 4096
HIST = 50
NC, NS, L = 2, 16, 16
NW = NC * NS
BPW = B // NW

_mesh = plsc.VectorSubcoreMesh(core_axis_name="c", subcore_axis_name="s")


@functools.partial(
    pl.kernel,
    mesh=_mesh,
    out_type=jax.ShapeDtypeStruct((B,), jnp.float32),
    scratch_types=(
        [
            pltpu.VMEM((HIST, BPW), jnp.int32),
            pltpu.VMEM((BPW,), jnp.int32),
            pltpu.VMEM((BPW, D), jnp.float32),
            pltpu.VMEM((BPW, D), jnp.float32),
            pltpu.VMEM((BPW,), jnp.float32),
            pltpu.VMEM((BPW,), jnp.float32),
        ]
        + [pltpu.SemaphoreType.DMA, pltpu.SemaphoreType.DMA]
    ),
    compiler_params=pltpu.CompilerParams(use_tc_tiling_on_sc=False),
)
def _bprmf_sc(seq_hbm, tgt_hbm, table_hbm, out_hbm,
              st, tgt_idx, tgt_rows, acc, wbuf, out_buf, gsem, tsem):
    wid = lax.axis_index("s") * NC + lax.axis_index("c")
    base = wid * BPW

    lane = lax.iota(jnp.int32, L)
    zero = jnp.zeros((L,), jnp.float32)
    one = jnp.ones((L,), jnp.float32)

    pltpu.sync_copy(seq_hbm.at[:, pl.ds(base, BPW)], st)
    pltpu.sync_copy(tgt_hbm.at[wid], tgt_idx)

    pltpu.async_copy(table_hbm.at[tgt_idx], tgt_rows, tsem)

    def _zbody(b, carry):
        for k in range(D // L):
            acc[b, pl.ds(k * L, L)] = zero
        return carry
    lax.fori_loop(0, BPW, _zbody, 0, unroll=8)

    descs = [pltpu.async_copy(table_hbm.at[st.at[l]], acc, gsem, add=True)
             for l in range(HIST)]

    def _cbody(t, carry):
        def _lbody(l, w):
            s = st[l, pl.ds(t * L, L)]
            return w + jnp.where(s != 0, one, zero)
        w = lax.fori_loop(0, HIST, _lbody, zero, unroll=4)
        wbuf[pl.ds(t * L, L)] = w
        return carry
    lax.fori_loop(0, BPW // L, _cbody, 0)

    pltpu.make_async_copy(table_hbm.at[tgt_idx], tgt_rows, tsem).wait()
    for d in descs:
        d.wait()

    def _allreduce_sum(v):
        for k in (8, 4, 2, 1):
            v = v + v.at[lane ^ k].get(mode="promise_in_bounds")
        return v

    def _obody(t, carry):
        res = zero
        for j in range(L):
            b = t * L + j
            dotv = zero
            for k in range(D // L):
                dotv = dotv + (acc[b, pl.ds(k * L, L)]
                               * tgt_rows[b, pl.ds(k * L, L)])
            pred_v = _allreduce_sum(dotv)
            res = jnp.where(lane == j, pred_v, res)
        w = wbuf[pl.ds(t * L, L)]
        out_buf[pl.ds(t * L, L)] = res / w
        return carry
    lax.fori_loop(0, BPW // L, _obody, 0)

    pltpu.sync_copy(out_buf, out_hbm.at[pl.ds(base, BPW)])


def kernel(seq, target, embed_weight):
    seq_t = jnp.swapaxes(seq.astype(jnp.int32), 0, 1)
    tgt_w = target.astype(jnp.int32).reshape(NW, BPW)
    return _bprmf_sc(seq_t, tgt_w, embed_weight)
